# baseline (speedup 1.0000x reference)
import functools
import math

import jax
import jax.numpy as jnp
from jax import lax
from jax.experimental import pallas as pl
from jax.experimental.pallas import tpu as pltpu
from jax.experimental.pallas import tpu_sc as plsc

D_MODEL = 64
SCALE = math.sqrt(D_MODEL)

_NC = 2
_NS = 16
_NW = _NC * _NS
_CHUNK = 128
_LANES = 16
_NBUF = 4


@functools.lru_cache(maxsize=None)
def _make_lookup_kernel(B, V):
    assert B % (_NW * _CHUNK * _NBUF) == 0
    rows_per_w = B // _NW
    nch = rows_per_w // _CHUNK

    mesh = plsc.VectorSubcoreMesh(core_axis_name="c", subcore_axis_name="s")

    @functools.partial(
        pl.kernel,
        mesh=mesh,
        compiler_params=pltpu.CompilerParams(needs_layout_passes=False),
        out_type=jax.ShapeDtypeStruct((B, D_MODEL), jnp.float32),
        scratch_types=[
            pltpu.VMEM((nch, _CHUNK), jnp.int32),
            pltpu.VMEM((_NBUF, _CHUNK), jnp.int32),
            pltpu.VMEM((_NBUF, _CHUNK, 2 * D_MODEL), jnp.float32),
            pltpu.VMEM((2, _CHUNK, D_MODEL), jnp.float32),
            pltpu.SemaphoreType.DMA,
            pltpu.SemaphoreType.DMA,
            pltpu.SemaphoreType.DMA,
            pltpu.SemaphoreType.DMA,
            pltpu.SemaphoreType.DMA,
            pltpu.SemaphoreType.DMA,
        ],
    )
    def k2(x_hbm, t2_hbm, out_hbm, idx_v, pair_v, in_v, out_v,
           g0, g1, g2, g3, o0, o1):
        gsems = (g0, g1, g2, g3)
        osems = (o0, o1)
        wid = lax.axis_index("s") * _NC + lax.axis_index("c")
        base_idx_row = wid * nch
        base_out = wid * rows_per_w
        pltpu.sync_copy(x_hbm.at[pl.ds(base_idx_row, nch)], idx_v)

        def start_gather(j, b):
            def pair_body(kk, c):
                sl = pl.ds(kk * _LANES, _LANES)
                pair_v[b, sl] = lax.shift_right_logical(idx_v[j, sl], 1)
                return c

            lax.fori_loop(0, _CHUNK // _LANES, pair_body, 0)
            pltpu.async_copy(t2_hbm.at[pair_v.at[b]], in_v.at[b], gsems[b])

        start_gather(0, 0)
        start_gather(1, 1)

        def process_chunk(j, b, ob):
            @pl.when(j + 2 < nch)
            def _():
                start_gather(j + 2, (b + 2) % _NBUF)

            pltpu.make_async_copy(
                t2_hbm.at[pair_v.at[b]], in_v.at[b], gsems[b]
            ).wait()

            @pl.when(j >= 2)
            def _():
                pltpu.make_async_copy(
                    out_v.at[ob], out_hbm.at[pl.ds(base_out, _CHUNK)], osems[ob]
                ).wait()

            def group_body(g, c):
                idxv = idx_v[j, pl.ds(g * _LANES, _LANES)]
                par = idxv & 1
                masks = [
                    jnp.full((_LANES,), par[ll], jnp.int32) > 0
                    for ll in range(_LANES)
                ]
                for ll in range(_LANES):
                    r = g * _LANES + ll
                    for kk in range(D_MODEL // _LANES):
                        o = kk * _LANES
                        lo = in_v[b, r, pl.ds(o, _LANES)]
                        hi = in_v[b, r, pl.ds(D_MODEL + o, _LANES)]
                        out_v[ob, r, pl.ds(o, _LANES)] = (
                            jnp.where(masks[ll], hi, lo) * SCALE
                        )
                return c

            lax.fori_loop(0, _CHUNK // _LANES, group_body, 0)
            pltpu.async_copy(
                out_v.at[ob],
                out_hbm.at[pl.ds(base_out + j * _CHUNK, _CHUNK)],
                osems[ob],
            )

        def outer_body(jj, carry):
            for u in range(_NBUF):
                process_chunk(_NBUF * jj + u, u, u % 2)
            return carry

        lax.fori_loop(0, nch // _NBUF, outer_body, 0)
        for ob in range(2):
            pltpu.make_async_copy(
                out_v.at[ob], out_hbm.at[pl.ds(base_out, _CHUNK)], osems[ob]
            ).wait()

    return k2


def kernel(x, table):
    B = x.size
    V = table.shape[0]
    x2 = x.reshape(-1, _CHUNK).astype(jnp.int32)
    t2 = jnp.concatenate([table[0::2], table[1::2]], axis=1)
    out = _make_lookup_kernel(B, V)(x2, t2)
    return out.reshape(x.shape + (D_MODEL,))


# speedup vs baseline: 8.5909x; 8.5909x over previous
import functools
import math

import jax
import jax.numpy as jnp
from jax import lax
from jax.experimental import pallas as pl
from jax.experimental.pallas import tpu as pltpu
from jax.experimental.pallas import tpu_sc as plsc

D_MODEL = 64
SCALE = math.sqrt(D_MODEL)

_NC = 2
_NS = 16
_NW = _NC * _NS
_CHUNK = 128
_LANES = 16
_NBUF = 4


@functools.lru_cache(maxsize=None)
def _make_lookup_kernel(B, V):
    assert B % (_NW * _CHUNK * _NBUF) == 0
    rows_per_w = B // _NW
    nch = rows_per_w // _CHUNK

    mesh = plsc.VectorSubcoreMesh(core_axis_name="c", subcore_axis_name="s")

    @functools.partial(
        pl.kernel,
        mesh=mesh,
        compiler_params=pltpu.CompilerParams(needs_layout_passes=False),
        out_type=jax.ShapeDtypeStruct((B, D_MODEL), jnp.float32),
        scratch_types=[
            pltpu.VMEM((nch, _CHUNK), jnp.int32),
            pltpu.VMEM((_NBUF, _CHUNK), jnp.int32),
            pltpu.VMEM((_NBUF, _CHUNK, 2 * D_MODEL), jnp.float32),
            pltpu.VMEM((2, _CHUNK, D_MODEL), jnp.float32),
            pltpu.SemaphoreType.DMA,
            pltpu.SemaphoreType.DMA,
            pltpu.SemaphoreType.DMA,
            pltpu.SemaphoreType.DMA,
            pltpu.SemaphoreType.DMA,
            pltpu.SemaphoreType.DMA,
        ],
    )
    def k2(x_hbm, t2_hbm, out_hbm, idx_v, pair_v, in_v, out_v,
           g0, g1, g2, g3, o0, o1):
        gsems = (g0, g1, g2, g3)
        osems = (o0, o1)
        wid = lax.axis_index("s") * _NC + lax.axis_index("c")
        base_idx_row = wid * nch
        base_out = wid * rows_per_w
        pltpu.sync_copy(x_hbm.at[pl.ds(base_idx_row, nch)], idx_v)

        def start_gather(j, b):
            def pair_body(kk, c):
                sl = pl.ds(kk * _LANES, _LANES)
                pair_v[b, sl] = lax.shift_right_logical(idx_v[j, sl], 1)
                return c

            lax.fori_loop(0, _CHUNK // _LANES, pair_body, 0)
            pltpu.async_copy(t2_hbm.at[pair_v.at[b]], in_v.at[b], gsems[b])

        start_gather(0, 0)
        start_gather(1, 1)

        def process_chunk(j, b, ob):
            @pl.when(j + 2 < nch)
            def _():
                start_gather(j + 2, (b + 2) % _NBUF)

            pltpu.make_async_copy(
                t2_hbm.at[pair_v.at[b]], in_v.at[b], gsems[b]
            ).wait()

            @pl.when(j >= 2)
            def _():
                pltpu.make_async_copy(
                    out_v.at[ob], out_hbm.at[pl.ds(base_out, _CHUNK)], osems[ob]
                ).wait()

            def group_body(g, c):
                idxv = idx_v[j, pl.ds(g * _LANES, _LANES)]
                par = idxv & 1
                masks = [
                    jnp.full((_LANES,), par[ll], jnp.int32) > 0
                    for ll in range(_LANES)
                ]
                for ll in range(_LANES):
                    r = g * _LANES + ll
                    for kk in range(D_MODEL // _LANES):
                        o = kk * _LANES
                        lo = in_v[b, r, pl.ds(o, _LANES)]
                        hi = in_v[b, r, pl.ds(D_MODEL + o, _LANES)]
                        out_v[ob, r, pl.ds(o, _LANES)] = (
                            jnp.where(masks[ll], hi, lo) * SCALE
                        )
                return c

            lax.fori_loop(0, _CHUNK // _LANES, group_body, 0)
            pltpu.async_copy(
                out_v.at[ob],
                out_hbm.at[pl.ds(base_out + j * _CHUNK, _CHUNK)],
                osems[ob],
            )

        def outer_body(jj, carry):
            for u in range(_NBUF):
                process_chunk(_NBUF * jj + u, u, u % 2)
            return carry

        lax.fori_loop(0, nch // _NBUF, outer_body, 0)
        for ob in range(2):
            pltpu.make_async_copy(
                out_v.at[ob], out_hbm.at[pl.ds(base_out, _CHUNK)], osems[ob]
            ).wait()

    return k2


def kernel(x, table):
    B = x.size
    V = table.shape[0]
    x2 = x.reshape(-1, _CHUNK).astype(jnp.int32)
    t2 = table.reshape(V // 2, 2 * D_MODEL)
    out = _make_lookup_kernel(B, V)(x2, t2)
    return out.reshape(x.shape + (D_MODEL,))
